# baseline (speedup 1.0000x reference)
import jax
import jax.numpy as jnp
from jax import lax
from jax.experimental import pallas as pl
from jax.experimental.pallas import tpu as pltpu

STRIDE_F = 8.0
N_CLS_K = 80
NB, NH, NW = 8, 128, 128
NLOC = NH * NW
NSPLIT = 8
R = NLOC // NSPLIT
BH = NH // NSPLIT


def _sigmoid(x):
    return 1.0 / (1.0 + jnp.exp(-x))


def _body(hw_ref, bbox_ref, center_ref, cls_ref, xywh_ref, idx_ref, conf_ref):
    r = pl.program_id(0)
    b = pl.program_id(1)

    logits = cls_ref[0].reshape(R, N_CLS_K)
    xt = logits.T
    m = jnp.max(xt, axis=0, keepdims=True)
    ii = lax.broadcasted_iota(jnp.int32, (N_CLS_K, R), 0)
    idx = jnp.min(jnp.where(xt == m, ii, N_CLS_K), axis=0, keepdims=True)
    idx_ref[pl.ds(b, 1), :] = idx

    c = center_ref[0].reshape(1, R)
    conf_ref[pl.ds(b, 1), :] = jnp.sqrt(_sigmoid(c) * _sigmoid(m))

    v = bbox_ref[0]
    ri = lax.broadcasted_iota(jnp.int32, (BH, NW, 4), 0)
    li = lax.broadcasted_iota(jnp.int32, (BH, NW, 4), 1)
    c2 = lax.broadcasted_iota(jnp.int32, (BH, NW, 4), 2)
    xf = li.astype(jnp.float32) * STRIDE_F + STRIDE_F / 2.0
    yf = (r * BH + ri).astype(jnp.float32) * STRIDE_F + STRIDE_F / 2.0
    is_x = (c2 & 1) == 0
    coord = jnp.where(is_x, xf, yf)
    w_f = hw_ref[1].astype(jnp.float32)
    h_f = hw_ref[0].astype(jnp.float32)
    bound = jnp.where(is_x, w_f, h_f)
    lo_half = c2 < 2
    sgn = jnp.where(lo_half, -1.0, 1.0)
    p = jnp.exp(v) * STRIDE_F
    cl = jnp.clip(coord + sgn * p, 0.0, bound)
    sw = jnp.roll(cl, 2, axis=2)
    out = jnp.where(lo_half, (cl + sw) * 0.5, cl - sw)
    xywh_ref[0] = out.reshape(R, 4)


def kernel(bbox, center, cls_logits, img_h, img_w):
    nB, nH, nW, _ = bbox.shape
    hw = jnp.stack([jnp.asarray(img_h, jnp.int32), jnp.asarray(img_w, jnp.int32)])

    grid = (NSPLIT, nB)
    xywh, idx, conf = pl.pallas_call(
        _body,
        grid=grid,
        in_specs=[
            pl.BlockSpec(memory_space=pltpu.SMEM),
            pl.BlockSpec((1, BH, NW, 4), lambda r, b: (b, r, 0, 0)),
            pl.BlockSpec((1, BH, NW, 1), lambda r, b: (b, r, 0, 0)),
            pl.BlockSpec((1, BH, NW, N_CLS_K), lambda r, b: (b, r, 0, 0)),
        ],
        out_specs=[
            pl.BlockSpec((1, R, 4), lambda r, b: (b, r, 0)),
            pl.BlockSpec((NB, R), lambda r, b: (0, r)),
            pl.BlockSpec((NB, R), lambda r, b: (0, r)),
        ],
        out_shape=[
            jax.ShapeDtypeStruct((nB, NLOC, 4), jnp.float32),
            jax.ShapeDtypeStruct((nB, NLOC), jnp.int32),
            jax.ShapeDtypeStruct((nB, NLOC), jnp.float32),
        ],
        compiler_params=pltpu.CompilerParams(
            dimension_semantics=("parallel", "arbitrary"),
        ),
    )(hw, bbox, center, cls_logits)

    return (xywh, idx, conf)


# speedup vs baseline: 15.5470x; 15.5470x over previous
import jax
import jax.numpy as jnp
from jax import lax
from jax.experimental import pallas as pl
from jax.experimental.pallas import tpu as pltpu

STRIDE_F = 8.0
N_CLS_K = 80
NB, NH, NW = 8, 128, 128
NLOC = NH * NW
NSPLIT = 1
R = NLOC // NSPLIT
BH = NH // NSPLIT


def _sigmoid(x):
    return 1.0 / (1.0 + jnp.exp(-x))


def _rows_to_lanes(x):
    return jnp.concatenate([x[g:g + 1, :] for g in range(BH)], axis=1)


def _body(hw_ref, bbox_ref, center_ref, cls_ref, xywh_ref, idx_ref, conf_ref):
    r = pl.program_id(0)
    b = pl.program_id(1)

    logits = cls_ref[0]
    m = jnp.max(logits, axis=1)
    ii = lax.broadcasted_iota(jnp.int32, (BH, N_CLS_K, NW), 1)
    hit = logits == m[:, None, :]
    idx = jnp.min(jnp.where(hit, ii, N_CLS_K), axis=1)
    idx_ref[pl.ds(b, 1), :] = _rows_to_lanes(idx)

    c = center_ref[0]
    conf = jnp.sqrt(_sigmoid(c) * _sigmoid(m))
    conf_ref[pl.ds(b, 1), :] = _rows_to_lanes(conf)

    v = bbox_ref[0]
    ri = lax.broadcasted_iota(jnp.int32, (BH, 4, NW), 0)
    li = lax.broadcasted_iota(jnp.int32, (BH, 4, NW), 2)
    c2 = lax.broadcasted_iota(jnp.int32, (BH, 4, NW), 1)
    xf = li.astype(jnp.float32) * STRIDE_F + STRIDE_F / 2.0
    yf = (r * BH + ri).astype(jnp.float32) * STRIDE_F + STRIDE_F / 2.0
    is_x = (c2 & 1) == 0
    coord = jnp.where(is_x, xf, yf)
    w_f = hw_ref[1].astype(jnp.float32)
    h_f = hw_ref[0].astype(jnp.float32)
    bound = jnp.where(is_x, w_f, h_f)
    lo_half = c2 < 2
    sgn = jnp.where(lo_half, -1.0, 1.0)
    p = jnp.exp(v) * STRIDE_F
    cl = jnp.clip(coord + sgn * p, 0.0, bound)
    sw = jnp.roll(cl, 2, axis=1)
    out = jnp.where(lo_half, (cl + sw) * 0.5, cl - sw)
    chans = [
        jnp.concatenate([out[g, cc][None, :] for g in range(BH)], axis=1)
        for cc in range(4)
    ]
    xywh_ref[0] = jnp.concatenate(chans, axis=0)


def kernel(bbox, center, cls_logits, img_h, img_w):
    nB, nH, nW, _ = bbox.shape
    bbox_t = bbox.transpose(0, 1, 3, 2)
    cls_t = cls_logits.transpose(0, 1, 3, 2)
    center_sq = center.reshape(nB, nH, nW)
    hw = jnp.stack([jnp.asarray(img_h, jnp.int32), jnp.asarray(img_w, jnp.int32)])

    grid = (NSPLIT, nB)
    xywh_t, idx, conf = pl.pallas_call(
        _body,
        grid=grid,
        in_specs=[
            pl.BlockSpec(memory_space=pltpu.SMEM),
            pl.BlockSpec((1, BH, 4, NW), lambda r, b: (b, r, 0, 0)),
            pl.BlockSpec((1, BH, NW), lambda r, b: (b, r, 0)),
            pl.BlockSpec((1, BH, N_CLS_K, NW), lambda r, b: (b, r, 0, 0)),
        ],
        out_specs=[
            pl.BlockSpec((1, 4, R), lambda r, b: (b, 0, r)),
            pl.BlockSpec((NB, R), lambda r, b: (0, r)),
            pl.BlockSpec((NB, R), lambda r, b: (0, r)),
        ],
        out_shape=[
            jax.ShapeDtypeStruct((nB, 4, NLOC), jnp.float32),
            jax.ShapeDtypeStruct((nB, NLOC), jnp.int32),
            jax.ShapeDtypeStruct((nB, NLOC), jnp.float32),
        ],
        compiler_params=pltpu.CompilerParams(
            dimension_semantics=("parallel", "arbitrary"),
        ),
    )(hw, bbox_t, center_sq, cls_t)

    return (xywh_t.transpose(0, 2, 1), idx, conf)
